# baseline (speedup 1.0000x reference)
import jax
import jax.numpy as jnp
from jax.experimental import pallas as pl
from jax.experimental.pallas import tpu as pltpu

_NB = 5


def _matcher_kernel(ts_ref, te_ref, es_ref, ee_ref, out_ref, vmax_ref, vidx_ref):
    j = pl.program_id(1)
    nb = pl.num_programs(1)
    ws = jnp.exp(ts_ref[0])
    we = jnp.exp(te_ref[0])
    es = es_ref[0]
    ee = ee_ref[0]
    dn = (((1,), (1,)), ((), ()))
    s = jax.lax.dot_general(ws, es, dn, precision=jax.lax.Precision.HIGHEST,
                            preferred_element_type=jnp.float32)
    s = s + jax.lax.dot_general(we, ee, dn, precision=jax.lax.Precision.HIGHEST,
                                preferred_element_type=jnp.float32)
    mx = jnp.max(s, axis=1, keepdims=True)
    iota = jax.lax.broadcasted_iota(jnp.int32, s.shape, 1)
    idx = jnp.min(jnp.where(s == mx, iota, 2**30), axis=1, keepdims=True)
    idx = idx + j * s.shape[1]

    @pl.when(j == 0)
    def _init():
        vmax_ref[...] = mx
        vidx_ref[...] = idx

    @pl.when(j > 0)
    def _merge():
        better = mx > vmax_ref[...]
        vidx_ref[...] = jnp.where(better, idx, vidx_ref[...])
        vmax_ref[...] = jnp.maximum(mx, vmax_ref[...])

    @pl.when(j == nb - 1)
    def _finish():
        out_ref[0, 0, :] = vidx_ref[:, 0]


def kernel(ent_start_probs, ent_end_probs, ent_part_probs,
           target_start_probs, target_end_probs, target_part_probs):
    B, N, L = ent_start_probs.shape
    M = target_start_probs.shape[1]
    nblk = N // _NB
    out = pl.pallas_call(
        _matcher_kernel,
        grid=(B, _NB),
        in_specs=[
            pl.BlockSpec((1, M, L), lambda b, j: (b, 0, 0)),
            pl.BlockSpec((1, M, L), lambda b, j: (b, 0, 0)),
            pl.BlockSpec((1, nblk, L), lambda b, j: (b, j, 0)),
            pl.BlockSpec((1, nblk, L), lambda b, j: (b, j, 0)),
        ],
        out_specs=pl.BlockSpec((1, 1, M), lambda b, j: (b, 0, 0)),
        out_shape=jax.ShapeDtypeStruct((B, 1, M), jnp.int32),
        scratch_shapes=[
            pltpu.VMEM((M, 1), jnp.float32),
            pltpu.VMEM((M, 1), jnp.int32),
        ],
    )(target_start_probs, target_end_probs, ent_start_probs, ent_end_probs)
    return out.reshape(B, M)


# speedup vs baseline: 1.2804x; 1.2804x over previous
import jax
import jax.numpy as jnp
from jax.experimental import pallas as pl
from jax.experimental.pallas import tpu as pltpu

_KB = 2


def _matcher_kernel(ts_ref, te_ref, es_ref, ee_ref, out_ref, acc_ref):
    k = pl.program_id(1)
    nk = pl.num_programs(1)
    ws = jnp.exp(ts_ref[0])
    we = jnp.exp(te_ref[0])
    es = es_ref[0]
    ee = ee_ref[0]
    dn = (((1,), (1,)), ((), ()))
    s = jax.lax.dot_general(ws, es, dn, precision=jax.lax.Precision.HIGHEST,
                            preferred_element_type=jnp.float32)
    s = s + jax.lax.dot_general(we, ee, dn, precision=jax.lax.Precision.HIGHEST,
                                preferred_element_type=jnp.float32)

    @pl.when(k == 0)
    def _init():
        acc_ref[...] = s

    @pl.when(k > 0)
    def _acc():
        acc_ref[...] += s

    @pl.when(k == nk - 1)
    def _finish():
        st = acc_ref[...]
        mx = jnp.max(st, axis=1, keepdims=True)
        iota = jax.lax.broadcasted_iota(jnp.int32, st.shape, 1)
        idx = jnp.min(jnp.where(st == mx, iota, 2**30), axis=1)
        out_ref[0, 0, :] = idx


def kernel(ent_start_probs, ent_end_probs, ent_part_probs,
           target_start_probs, target_end_probs, target_part_probs):
    B, N, L = ent_start_probs.shape
    M = target_start_probs.shape[1]
    lb = L // _KB
    out = pl.pallas_call(
        _matcher_kernel,
        grid=(B, _KB),
        in_specs=[
            pl.BlockSpec((1, M, lb), lambda b, k: (b, 0, k)),
            pl.BlockSpec((1, M, lb), lambda b, k: (b, 0, k)),
            pl.BlockSpec((1, N, lb), lambda b, k: (b, 0, k)),
            pl.BlockSpec((1, N, lb), lambda b, k: (b, 0, k)),
        ],
        out_specs=pl.BlockSpec((1, 1, M), lambda b, k: (b, 0, 0)),
        out_shape=jax.ShapeDtypeStruct((B, 1, M), jnp.int32),
        scratch_shapes=[pltpu.VMEM((M, N), jnp.float32)],
    )(target_start_probs, target_end_probs, ent_start_probs, ent_end_probs)
    return out.reshape(B, M)


# speedup vs baseline: 1.4595x; 1.1399x over previous
import jax
import jax.numpy as jnp
from jax.experimental import pallas as pl


def _matcher_kernel(ts_ref, te_ref, es_ref, ee_ref, out_ref):
    ws = jnp.exp(ts_ref[0])
    we = jnp.exp(te_ref[0])
    es = es_ref[0]
    ee = ee_ref[0]
    dn = (((1,), (1,)), ((), ()))
    st = jax.lax.dot_general(es, ws, dn, precision=jax.lax.Precision.HIGHEST,
                             preferred_element_type=jnp.float32)
    st = st + jax.lax.dot_general(ee, we, dn, precision=jax.lax.Precision.HIGHEST,
                                  preferred_element_type=jnp.float32)
    mx = jnp.max(st, axis=0, keepdims=True)
    iota = jax.lax.broadcasted_iota(jnp.int32, st.shape, 0)
    idx = jnp.min(jnp.where(st == mx, iota, 2**30), axis=0)
    out_ref[0, 0, :] = idx


def kernel(ent_start_probs, ent_end_probs, ent_part_probs,
           target_start_probs, target_end_probs, target_part_probs):
    B, N, L = ent_start_probs.shape
    M = target_start_probs.shape[1]
    out = pl.pallas_call(
        _matcher_kernel,
        grid=(B,),
        in_specs=[
            pl.BlockSpec((1, M, L), lambda i: (i, 0, 0)),
            pl.BlockSpec((1, M, L), lambda i: (i, 0, 0)),
            pl.BlockSpec((1, N, L), lambda i: (i, 0, 0)),
            pl.BlockSpec((1, N, L), lambda i: (i, 0, 0)),
        ],
        out_specs=pl.BlockSpec((1, 1, M), lambda i: (i, 0, 0)),
        out_shape=jax.ShapeDtypeStruct((B, 1, M), jnp.int32),
    )(target_start_probs, target_end_probs, ent_start_probs, ent_end_probs)
    return out.reshape(B, M)
